# baseline (speedup 1.0000x reference)
import functools

import jax
import jax.numpy as jnp
from jax import lax
from jax.experimental import pallas as pl
from jax.experimental.pallas import tpu as pltpu
from jax.experimental.pallas import tpu_sc as plsc

B = 8
M = 2048
K = 8
NC = 2
NS = 16
NW = NC * NS


def _fps_body(pc_ref, out_ref):
    def step(i, carry):
        distance, far, buf = carry
        iota = lax.broadcasted_iota(jnp.int32, (B, M), 1)
        lane = lax.broadcasted_iota(jnp.int32, (B, 128), 1)
        li = jnp.bitwise_and(i, 127)
        buf = jnp.where(lane == li, far, buf)

        @pl.when(li == 127)
        def _():
            base = pl.multiple_of(i - 127, 128)
            out_ref[:, pl.ds(base, 128)] = buf

        sel = iota == far
        x = pc_ref[:, 0, :]
        y = pc_ref[:, 1, :]
        z = pc_ref[:, 2, :]
        cx = jnp.sum(jnp.where(sel, x, 0.0), axis=1, keepdims=True)
        cy = jnp.sum(jnp.where(sel, y, 0.0), axis=1, keepdims=True)
        cz = jnp.sum(jnp.where(sel, z, 0.0), axis=1, keepdims=True)
        dx = x - cx
        dy = y - cy
        dz = z - cz
        dist = dx * dx + dy * dy + dz * dz
        distance = jnp.minimum(distance, dist)
        m = jnp.max(distance, axis=1, keepdims=True)
        far = jnp.min(jnp.where(distance == m, iota, M), axis=1, keepdims=True)
        return distance, far, buf

    x0 = pc_ref[:, 0, :]
    init = (x0 * 0.0 + 1e10,
            (x0[:, 0:1] * 0.0).astype(jnp.int32),
            (x0[:, 0:128] * 0.0).astype(jnp.int32))
    lax.fori_loop(0, M, step, init)


def _fps(pc):
    return pl.pallas_call(
        _fps_body,
        out_shape=jax.ShapeDtypeStruct((B, M), jnp.int32),
    )(pc)


_RB = 256


def _knn_body(xyzt_ref, pc_ref, out_ref):
    xrows = xyzt_ref[0]
    x0 = pc_ref[0, 0:1, :]
    x1 = pc_ref[0, 1:2, :]
    x2 = pc_ref[0, 2:3, :]
    nall = x0 * x0 + x1 * x1 + x2 * x2
    nrows = jnp.sum(xrows * xrows, axis=1, keepdims=True)
    cross = jnp.dot(xrows, pc_ref[0], preferred_element_type=jnp.float32)
    d = (nrows + nall) - 2.0 * cross
    iota = lax.broadcasted_iota(jnp.int32, (_RB, M), 1)
    cols = []
    for _ in range(K):
        m = jnp.min(d, axis=1, keepdims=True)
        j = jnp.min(jnp.where(d == m, iota, M), axis=1, keepdims=True)
        cols.append(j)
        d = jnp.where(iota == j, jnp.float32(3e38), d)
    out_ref[0] = jnp.concatenate(cols, axis=1)


def _knn(xyzt, pc):
    grid = (B, M // _RB)
    return pl.pallas_call(
        _knn_body,
        grid=grid,
        in_specs=[
            pl.BlockSpec((1, _RB, 3), lambda b, r: (b, r, 0)),
            pl.BlockSpec((1, 3, M), lambda b, r: (b, 0, 0)),
        ],
        out_specs=pl.BlockSpec((1, _RB, K), lambda b, r: (b, r, 0)),
        out_shape=jax.ShapeDtypeStruct((B, M, K), jnp.int32),
    )(xyzt, pc)


def _sc_mesh():
    return plsc.VectorSubcoreMesh(core_axis_name="c", subcore_axis_name="s",
                                  num_cores=NC, num_subcores=NS)


def _wid():
    return lax.axis_index("s") * NC + lax.axis_index("c")


def _add_offset(idx_v, n, off):
    def body(j, _):
        idx_v[pl.ds(j * 16, 16)] = idx_v[pl.ds(j * 16, 16)] + off
        return 0
    lax.fori_loop(0, n // 16, body, 0, unroll=4)


def _sc_gather_a(xyzpad, knn, fps_flat):
    rows = (B * M) // NW

    def body(xyz_hbm, knn_hbm, fps_hbm, nxyz_hbm, knng_hbm,
             idx_v, rows_v, irows_v, sem):
        w = _wid()
        base = w * rows
        off = (base // M) * M
        pltpu.sync_copy(fps_hbm.at[pl.ds(base, rows)], idx_v)
        _add_offset(idx_v, rows, off)
        pltpu.async_copy(xyz_hbm.at[idx_v], rows_v, sem).wait()
        pltpu.sync_copy(rows_v, nxyz_hbm.at[pl.ds(base, rows)])
        pltpu.async_copy(knn_hbm.at[idx_v], irows_v, sem).wait()
        pltpu.sync_copy(irows_v, knng_hbm.at[pl.ds(base, rows)])

    f = pl.kernel(
        body,
        out_type=(jax.ShapeDtypeStruct((B * M, 16), jnp.float32),
                  jax.ShapeDtypeStruct((B * M, K), jnp.int32)),
        mesh=_sc_mesh(),
        compiler_params=pltpu.CompilerParams(use_tc_tiling_on_sc=False),
        scratch_types=[
            pltpu.VMEM((rows,), jnp.int32),
            pltpu.VMEM((rows, 16), jnp.float32),
            pltpu.VMEM((rows, K), jnp.int32),
            pltpu.SemaphoreType.DMA,
        ],
    )
    return f(xyzpad, knn, fps_flat)


def _sc_gather_rows(table, idx_flat, width, chunk):
    total = B * M * K
    per_w = total // NW
    nchunk = per_w // chunk

    def body(tab_hbm, idx_hbm, out_hbm, idx_v, rows_v, sem):
        w = _wid()
        base = w * per_w
        off = (base // (M * K)) * M

        def chunk_body(c, _):
            cbase = base + c * chunk
            pltpu.sync_copy(idx_hbm.at[pl.ds(cbase, chunk)], idx_v)
            _add_offset(idx_v, chunk, off)
            pltpu.async_copy(tab_hbm.at[idx_v], rows_v, sem).wait()
            pltpu.sync_copy(rows_v, out_hbm.at[pl.ds(cbase, chunk)])
            return 0

        lax.fori_loop(0, nchunk, chunk_body, 0)

    f = pl.kernel(
        body,
        out_type=jax.ShapeDtypeStruct((total, width), jnp.float32),
        mesh=_sc_mesh(),
        compiler_params=pltpu.CompilerParams(use_tc_tiling_on_sc=False),
        scratch_types=[
            pltpu.VMEM((chunk,), jnp.int32),
            pltpu.VMEM((chunk, width), jnp.float32),
            pltpu.SemaphoreType.DMA,
        ],
    )
    return f(table, idx_flat)


def _stats(h):
    s = jnp.sum(h, axis=0, keepdims=True)
    mean = s / (M * K)
    v = jnp.sum((h - mean) * (h - mean), axis=0, keepdims=True)
    var = v / (M * K)
    return mean, var


def _affine(mean, var, g, be):
    scale = g * lax.rsqrt(var + 1e-5)
    shift = be - mean * scale
    return scale, shift


def _layer1_body(g1_ref, nxyz_ref, w0_ref, b0_ref, g0_ref, e0_ref,
                 w1_ref, b1_ref, g1w_ref, e1_ref,
                 w2_ref, b2_ref, g2w_ref, e2_ref,
                 out_ref, h1_ref, h2_ref):
    cen = nxyz_ref[0]
    cexp = jnp.broadcast_to(cen[:, None, :], (M, K, 16)).reshape(M * K, 16)
    gfull = g1_ref[0]
    x = jnp.concatenate([gfull[:, 0:16] - cexp, gfull[:, 16:80]], axis=1)
    h = jnp.dot(x, w0_ref[...], preferred_element_type=jnp.float32) + b0_ref[...]
    h1_ref[...] = h
    mean, var = _stats(h)
    scale, shift = _affine(mean, var, g0_ref[...], e0_ref[...])
    x = jnp.maximum(h1_ref[...] * scale + shift, 0.0)
    h = jnp.dot(x, w1_ref[...], preferred_element_type=jnp.float32) + b1_ref[...]
    h2_ref[...] = h
    mean, var = _stats(h)
    scale, shift = _affine(mean, var, g1w_ref[...], e1_ref[...])
    x = jnp.maximum(h2_ref[...] * scale + shift, 0.0)
    h = jnp.dot(x, w2_ref[...], preferred_element_type=jnp.float32) + b2_ref[...]
    h1_ref[...] = h
    mean, var = _stats(h)
    scale, shift = _affine(mean, var, g2w_ref[...], e2_ref[...])
    hn = jnp.maximum(h1_ref[...] * scale + shift, 0.0).reshape(M, K, 128)
    out_ref[0] = jnp.max(hn, axis=1)


def _layer1(g1, nxyz, w0p, b0, g0, e0, w1, b1, g1w, e1, w2, b2, g2w, e2):
    def vspec(shape):
        return pl.BlockSpec(shape, lambda b: (0,) * len(shape))
    return pl.pallas_call(
        _layer1_body,
        grid=(B,),
        in_specs=[
            pl.BlockSpec((1, M * K, 80), lambda b: (b, 0, 0)),
            pl.BlockSpec((1, M, 16), lambda b: (b, 0, 0)),
            vspec((80, 128)), vspec((1, 128)), vspec((1, 128)), vspec((1, 128)),
            vspec((128, 128)), vspec((1, 128)), vspec((1, 128)), vspec((1, 128)),
            vspec((128, 128)), vspec((1, 128)), vspec((1, 128)), vspec((1, 128)),
        ],
        out_specs=pl.BlockSpec((1, M, 128), lambda b: (b, 0, 0)),
        out_shape=jax.ShapeDtypeStruct((B, M, 128), jnp.float32),
        scratch_shapes=[
            pltpu.VMEM((M * K, 128), jnp.float32),
            pltpu.VMEM((M * K, 128), jnp.float32),
        ],
        compiler_params=pltpu.CompilerParams(vmem_limit_bytes=110 * 2**20),
    )(g1, nxyz, w0p, b0, g0, e0, w1, b1, g1w, e1, w2, b2, g2w, e2)


def _layer2_body(g1_ref, g2_ref, nxyz_ref, w_ref, b_ref, g_ref, e_ref,
                 out_ref, h_ref):
    cen = nxyz_ref[0]
    cexp = jnp.broadcast_to(cen[:, None, :], (M, K, 16)).reshape(M * K, 16)
    x = jnp.concatenate([g1_ref[0][:, 0:16] - cexp, g2_ref[0]], axis=1)
    h = jnp.dot(x, w_ref[...], preferred_element_type=jnp.float32) + b_ref[...]
    h_ref[...] = h
    mean, var = _stats(h)
    scale, shift = _affine(mean, var, g_ref[...], e_ref[...])
    hn = (h_ref[...] * scale + shift).reshape(M, K, 128)
    out_ref[0] = jnp.max(hn, axis=1)


def _layer2(g1, g2, nxyz, wp, b, g, e):
    def vspec(shape):
        return pl.BlockSpec(shape, lambda b: (0,) * len(shape))
    return pl.pallas_call(
        _layer2_body,
        grid=(B,),
        in_specs=[
            pl.BlockSpec((1, M * K, 80), lambda b: (b, 0, 0)),
            pl.BlockSpec((1, M * K, 128), lambda b: (b, 0, 0)),
            pl.BlockSpec((1, M, 16), lambda b: (b, 0, 0)),
            vspec((144, 128)), vspec((1, 128)), vspec((1, 128)), vspec((1, 128)),
        ],
        out_specs=pl.BlockSpec((1, M, 128), lambda b: (b, 0, 0)),
        out_shape=jax.ShapeDtypeStruct((B, M, 128), jnp.float32),
        scratch_shapes=[pltpu.VMEM((M * K, 128), jnp.float32)],
        compiler_params=pltpu.CompilerParams(vmem_limit_bytes=110 * 2**20),
    )(g1, g2, nxyz, wp, b, g, e)


def kernel(pc, feature, W1_0, b1_0, g1_0, be1_0, W1_1, b1_1, g1_1, be1_1,
           W1_2, b1_2, g1_2, be1_2, W2_0, b2_0, g2_0, be2_0):
    xyzt = jnp.transpose(pc, (0, 2, 1))
    pts = jnp.transpose(feature, (0, 2, 1))

    fps_idx = _fps(pc)
    knn = _knn(xyzt, pc)

    xyzpad = jnp.pad(xyzt, ((0, 0), (0, 0), (0, 13))).reshape(B * M, 16)
    nxyz_flat, knn_g = _sc_gather_a(xyzpad, knn.reshape(B * M, K),
                                    fps_idx.reshape(B * M))
    nxyz = nxyz_flat.reshape(B, M, 16)
    idx_flat = knn_g.reshape(B * M * K)

    table1 = jnp.concatenate([xyzpad.reshape(B, M, 16), pts], axis=2)
    g1 = _sc_gather_rows(table1.reshape(B * M, 80), idx_flat, 80, 512)
    g1 = g1.reshape(B, M * K, 80)

    def pack1(w):
        z = jnp.zeros((80, 128), jnp.float32)
        return lax.dynamic_update_slice(
            lax.dynamic_update_slice(z, w[0:3], (0, 0)), w[3:67], (16, 0))

    w0p = pack1(W1_0)
    r = lambda v: v.reshape(1, 128)
    feat_l1 = _layer1(g1, nxyz, w0p, r(b1_0), r(g1_0), r(be1_0),
                      W1_1, r(b1_1), r(g1_1), r(be1_1),
                      W1_2, r(b1_2), r(g1_2), r(be1_2))

    g2 = _sc_gather_rows(feat_l1.reshape(B * M, 128), idx_flat, 128, 512)
    g2 = g2.reshape(B, M * K, 128)

    z = jnp.zeros((144, 128), jnp.float32)
    w2p = lax.dynamic_update_slice(
        lax.dynamic_update_slice(z, W2_0[0:3], (0, 0)), W2_0[3:131], (16, 0))
    feat_l2 = _layer2(g1, g2, nxyz, w2p, r(b2_0), r(g2_0), r(be2_0))
    return jnp.transpose(feat_l2, (0, 2, 1))


# speedup vs baseline: 1.1162x; 1.1162x over previous
import functools

import jax
import jax.numpy as jnp
from jax import lax
from jax.experimental import pallas as pl
from jax.experimental.pallas import tpu as pltpu
from jax.experimental.pallas import tpu_sc as plsc

B = 8
M = 2048
K = 8
NC = 2
NS = 16
NW = NC * NS


def _fps_body(pc_ref, out_ref):
    sub = lax.broadcasted_iota(jnp.int32, (8, 256), 0)
    lan = lax.broadcasted_iota(jnp.int32, (8, 256), 1)

    def step(i, carry):
        dists, fars, accs = carry
        iota = sub * 256 + lan
        new_d, new_f, new_a = [], [], []
        for b in range(B):
            d, far, acc = dists[b], fars[b], accs[b]
            acc = jnp.where(iota == i, far, acc)
            sel = iota == far
            x = pc_ref[b, 0]
            y = pc_ref[b, 1]
            z = pc_ref[b, 2]
            cx = jnp.max(jnp.where(sel, x, -3e38), axis=1, keepdims=True)
            cy = jnp.max(jnp.where(sel, y, -3e38), axis=1, keepdims=True)
            cz = jnp.max(jnp.where(sel, z, -3e38), axis=1, keepdims=True)
            cx = jnp.max(cx, axis=0, keepdims=True)
            cy = jnp.max(cy, axis=0, keepdims=True)
            cz = jnp.max(cz, axis=0, keepdims=True)
            dx = x - cx
            dy = y - cy
            dz = z - cz
            dist = dx * dx + dy * dy + dz * dz
            d = jnp.minimum(d, dist)
            m = jnp.max(jnp.max(d, axis=1, keepdims=True), axis=0, keepdims=True)
            far = jnp.min(jnp.min(jnp.where(d == m, iota, M), axis=1,
                                  keepdims=True), axis=0, keepdims=True)
            new_d.append(d)
            new_f.append(far)
            new_a.append(acc)
        return tuple(new_d), tuple(new_f), tuple(new_a)

    dists, fars, accs = [], [], []
    for b in range(B):
        x0 = pc_ref[b, 0]
        dists.append(x0 * 0.0 + 1e10)
        fars.append((x0[0:1, 0:1] * 0.0).astype(jnp.int32))
        accs.append((x0 * 0.0).astype(jnp.int32))
    _, _, accs = lax.fori_loop(0, M, step, (tuple(dists), tuple(fars), tuple(accs)),
                               unroll=2)
    for b in range(B):
        out_ref[b] = accs[b]


def _fps(pc):
    pc4 = pc.reshape(B, 3, 8, 256)
    out = pl.pallas_call(
        _fps_body,
        out_shape=jax.ShapeDtypeStruct((B, 8, 256), jnp.int32),
    )(pc4)
    return out.reshape(B, M)


_RB = 256


def _knn_body(xyzt_ref, pc_ref, out_ref):
    xrows = xyzt_ref[0]
    x0 = pc_ref[0, 0:1, :]
    x1 = pc_ref[0, 1:2, :]
    x2 = pc_ref[0, 2:3, :]
    nall = x0 * x0 + x1 * x1 + x2 * x2
    nrows = jnp.sum(xrows * xrows, axis=1, keepdims=True)
    cross = jnp.dot(xrows, pc_ref[0], preferred_element_type=jnp.float32)
    d = (nrows + nall) - 2.0 * cross
    iota = lax.broadcasted_iota(jnp.int32, (_RB, M), 1)
    cols = []
    for _ in range(K):
        m = jnp.min(d, axis=1, keepdims=True)
        j = jnp.min(jnp.where(d == m, iota, M), axis=1, keepdims=True)
        cols.append(j)
        d = jnp.where(iota == j, jnp.float32(3e38), d)
    out_ref[0] = jnp.concatenate(cols, axis=1)


def _knn(xyzt, pc):
    grid = (B, M // _RB)
    return pl.pallas_call(
        _knn_body,
        grid=grid,
        in_specs=[
            pl.BlockSpec((1, _RB, 3), lambda b, r: (b, r, 0)),
            pl.BlockSpec((1, 3, M), lambda b, r: (b, 0, 0)),
        ],
        out_specs=pl.BlockSpec((1, _RB, K), lambda b, r: (b, r, 0)),
        out_shape=jax.ShapeDtypeStruct((B, M, K), jnp.int32),
    )(xyzt, pc)


def _sc_mesh():
    return plsc.VectorSubcoreMesh(core_axis_name="c", subcore_axis_name="s",
                                  num_cores=NC, num_subcores=NS)


def _wid():
    return lax.axis_index("s") * NC + lax.axis_index("c")


def _add_offset(idx_v, n, off):
    def body(j, _):
        idx_v[pl.ds(j * 16, 16)] = idx_v[pl.ds(j * 16, 16)] + off
        return 0
    lax.fori_loop(0, n // 16, body, 0, unroll=4)


def _sc_gather_a(xyzpad, knn, fps_flat):
    rows = (B * M) // NW

    def body(xyz_hbm, knn_hbm, fps_hbm, nxyz_hbm, knng_hbm,
             idx_v, rows_v, irows_v, sem):
        w = _wid()
        base = w * rows
        off = (base // M) * M
        pltpu.sync_copy(fps_hbm.at[pl.ds(base, rows)], idx_v)
        _add_offset(idx_v, rows, off)
        pltpu.async_copy(xyz_hbm.at[idx_v], rows_v, sem).wait()
        pltpu.sync_copy(rows_v, nxyz_hbm.at[pl.ds(base, rows)])
        pltpu.async_copy(knn_hbm.at[idx_v], irows_v, sem).wait()
        pltpu.sync_copy(irows_v, knng_hbm.at[pl.ds(base, rows)])

    f = pl.kernel(
        body,
        out_type=(jax.ShapeDtypeStruct((B * M, 16), jnp.float32),
                  jax.ShapeDtypeStruct((B * M, K), jnp.int32)),
        mesh=_sc_mesh(),
        compiler_params=pltpu.CompilerParams(use_tc_tiling_on_sc=False),
        scratch_types=[
            pltpu.VMEM((rows,), jnp.int32),
            pltpu.VMEM((rows, 16), jnp.float32),
            pltpu.VMEM((rows, K), jnp.int32),
            pltpu.SemaphoreType.DMA,
        ],
    )
    return f(xyzpad, knn, fps_flat)


def _sc_gather_rows(table, idx_flat, width, chunk):
    total = B * M * K
    per_w = total // NW
    nchunk = per_w // chunk

    def body(tab_hbm, idx_hbm, out_hbm, idx_v, rows_v, sem):
        w = _wid()
        base = w * per_w
        off = (base // (M * K)) * M

        def chunk_body(c, _):
            cbase = base + c * chunk
            pltpu.sync_copy(idx_hbm.at[pl.ds(cbase, chunk)], idx_v)
            _add_offset(idx_v, chunk, off)
            pltpu.async_copy(tab_hbm.at[idx_v], rows_v, sem).wait()
            pltpu.sync_copy(rows_v, out_hbm.at[pl.ds(cbase, chunk)])
            return 0

        lax.fori_loop(0, nchunk, chunk_body, 0)

    f = pl.kernel(
        body,
        out_type=jax.ShapeDtypeStruct((total, width), jnp.float32),
        mesh=_sc_mesh(),
        compiler_params=pltpu.CompilerParams(use_tc_tiling_on_sc=False),
        scratch_types=[
            pltpu.VMEM((chunk,), jnp.int32),
            pltpu.VMEM((chunk, width), jnp.float32),
            pltpu.SemaphoreType.DMA,
        ],
    )
    return f(table, idx_flat)


def _stats(h):
    s = jnp.sum(h, axis=0, keepdims=True)
    mean = s / (M * K)
    v = jnp.sum((h - mean) * (h - mean), axis=0, keepdims=True)
    var = v / (M * K)
    return mean, var


def _affine(mean, var, g, be):
    scale = g * lax.rsqrt(var + 1e-5)
    shift = be - mean * scale
    return scale, shift


def _layer1_body(g1_ref, nxyz_ref, w0_ref, b0_ref, g0_ref, e0_ref,
                 w1_ref, b1_ref, g1w_ref, e1_ref,
                 w2_ref, b2_ref, g2w_ref, e2_ref,
                 out_ref, h1_ref, h2_ref):
    cen = nxyz_ref[0]
    cexp = jnp.broadcast_to(cen[:, None, :], (M, K, 16)).reshape(M * K, 16)
    gfull = g1_ref[0]
    x = jnp.concatenate([gfull[:, 0:16] - cexp, gfull[:, 16:80]], axis=1)
    h = jnp.dot(x, w0_ref[...], preferred_element_type=jnp.float32) + b0_ref[...]
    h1_ref[...] = h
    mean, var = _stats(h)
    scale, shift = _affine(mean, var, g0_ref[...], e0_ref[...])
    x = jnp.maximum(h1_ref[...] * scale + shift, 0.0)
    h = jnp.dot(x, w1_ref[...], preferred_element_type=jnp.float32) + b1_ref[...]
    h2_ref[...] = h
    mean, var = _stats(h)
    scale, shift = _affine(mean, var, g1w_ref[...], e1_ref[...])
    x = jnp.maximum(h2_ref[...] * scale + shift, 0.0)
    h = jnp.dot(x, w2_ref[...], preferred_element_type=jnp.float32) + b2_ref[...]
    h1_ref[...] = h
    mean, var = _stats(h)
    scale, shift = _affine(mean, var, g2w_ref[...], e2_ref[...])
    hn = jnp.maximum(h1_ref[...] * scale + shift, 0.0).reshape(M, K, 128)
    out_ref[0] = jnp.max(hn, axis=1)


def _layer1(g1, nxyz, w0p, b0, g0, e0, w1, b1, g1w, e1, w2, b2, g2w, e2):
    def vspec(shape):
        return pl.BlockSpec(shape, lambda b: (0,) * len(shape))
    return pl.pallas_call(
        _layer1_body,
        grid=(B,),
        in_specs=[
            pl.BlockSpec((1, M * K, 80), lambda b: (b, 0, 0)),
            pl.BlockSpec((1, M, 16), lambda b: (b, 0, 0)),
            vspec((80, 128)), vspec((1, 128)), vspec((1, 128)), vspec((1, 128)),
            vspec((128, 128)), vspec((1, 128)), vspec((1, 128)), vspec((1, 128)),
            vspec((128, 128)), vspec((1, 128)), vspec((1, 128)), vspec((1, 128)),
        ],
        out_specs=pl.BlockSpec((1, M, 128), lambda b: (b, 0, 0)),
        out_shape=jax.ShapeDtypeStruct((B, M, 128), jnp.float32),
        scratch_shapes=[
            pltpu.VMEM((M * K, 128), jnp.float32),
            pltpu.VMEM((M * K, 128), jnp.float32),
        ],
        compiler_params=pltpu.CompilerParams(vmem_limit_bytes=110 * 2**20),
    )(g1, nxyz, w0p, b0, g0, e0, w1, b1, g1w, e1, w2, b2, g2w, e2)


def _layer2_body(g1_ref, g2_ref, nxyz_ref, w_ref, b_ref, g_ref, e_ref,
                 out_ref, h_ref):
    cen = nxyz_ref[0]
    cexp = jnp.broadcast_to(cen[:, None, :], (M, K, 16)).reshape(M * K, 16)
    x = jnp.concatenate([g1_ref[0][:, 0:16] - cexp, g2_ref[0]], axis=1)
    h = jnp.dot(x, w_ref[...], preferred_element_type=jnp.float32) + b_ref[...]
    h_ref[...] = h
    mean, var = _stats(h)
    scale, shift = _affine(mean, var, g_ref[...], e_ref[...])
    hn = (h_ref[...] * scale + shift).reshape(M, K, 128)
    out_ref[0] = jnp.max(hn, axis=1)


def _layer2(g1, g2, nxyz, wp, b, g, e):
    def vspec(shape):
        return pl.BlockSpec(shape, lambda b: (0,) * len(shape))
    return pl.pallas_call(
        _layer2_body,
        grid=(B,),
        in_specs=[
            pl.BlockSpec((1, M * K, 80), lambda b: (b, 0, 0)),
            pl.BlockSpec((1, M * K, 128), lambda b: (b, 0, 0)),
            pl.BlockSpec((1, M, 16), lambda b: (b, 0, 0)),
            vspec((144, 128)), vspec((1, 128)), vspec((1, 128)), vspec((1, 128)),
        ],
        out_specs=pl.BlockSpec((1, M, 128), lambda b: (b, 0, 0)),
        out_shape=jax.ShapeDtypeStruct((B, M, 128), jnp.float32),
        scratch_shapes=[pltpu.VMEM((M * K, 128), jnp.float32)],
        compiler_params=pltpu.CompilerParams(vmem_limit_bytes=110 * 2**20),
    )(g1, g2, nxyz, wp, b, g, e)


def kernel(pc, feature, W1_0, b1_0, g1_0, be1_0, W1_1, b1_1, g1_1, be1_1,
           W1_2, b1_2, g1_2, be1_2, W2_0, b2_0, g2_0, be2_0):
    xyzt = jnp.transpose(pc, (0, 2, 1))
    pts = jnp.transpose(feature, (0, 2, 1))

    fps_idx = _fps(pc)
    knn = _knn(xyzt, pc)

    xyzpad = jnp.pad(xyzt, ((0, 0), (0, 0), (0, 13))).reshape(B * M, 16)
    nxyz_flat, knn_g = _sc_gather_a(xyzpad, knn.reshape(B * M, K),
                                    fps_idx.reshape(B * M))
    nxyz = nxyz_flat.reshape(B, M, 16)
    idx_flat = knn_g.reshape(B * M * K)

    table1 = jnp.concatenate([xyzpad.reshape(B, M, 16), pts], axis=2)
    g1 = _sc_gather_rows(table1.reshape(B * M, 80), idx_flat, 80, 512)
    g1 = g1.reshape(B, M * K, 80)

    def pack1(w):
        z = jnp.zeros((80, 128), jnp.float32)
        return lax.dynamic_update_slice(
            lax.dynamic_update_slice(z, w[0:3], (0, 0)), w[3:67], (16, 0))

    w0p = pack1(W1_0)
    r = lambda v: v.reshape(1, 128)
    feat_l1 = _layer1(g1, nxyz, w0p, r(b1_0), r(g1_0), r(be1_0),
                      W1_1, r(b1_1), r(g1_1), r(be1_1),
                      W1_2, r(b1_2), r(g1_2), r(be1_2))

    g2 = _sc_gather_rows(feat_l1.reshape(B * M, 128), idx_flat, 128, 512)
    g2 = g2.reshape(B, M * K, 128)

    z = jnp.zeros((144, 128), jnp.float32)
    w2p = lax.dynamic_update_slice(
        lax.dynamic_update_slice(z, W2_0[0:3], (0, 0)), W2_0[3:131], (16, 0))
    feat_l2 = _layer2(g1, g2, nxyz, w2p, r(b2_0), r(g2_0), r(be2_0))
    return jnp.transpose(feat_l2, (0, 2, 1))
